# baseline (speedup 1.0000x reference)
import functools

import jax
import jax.numpy as jnp
from jax import lax
from jax.experimental import pallas as pl
from jax.experimental.pallas import tpu as pltpu
from jax.experimental.pallas import tpu_sc as plsc


def _build_gather(B: int, D: int):
    info = plsc.get_sparse_core_info()
    NC, NS, L = info.num_cores, info.num_subcores, info.num_lanes
    NW = NC * NS
    assert B % (8 * NW) == 0 and D % L == 0
    b_per_w = B // NW
    CHUNK = 256
    assert b_per_w % CHUNK == 0
    NCHUNK = b_per_w // CHUNK
    assert NCHUNK % 2 == 1

    mesh = plsc.VectorSubcoreMesh(core_axis_name="c", subcore_axis_name="s")

    @functools.partial(
        pl.kernel,
        mesh=mesh,
        compiler_params=pltpu.CompilerParams(
            use_tc_tiling_on_sc=True, needs_layout_passes=False
        ),
        out_type=jax.ShapeDtypeStruct((B, D), jnp.float32),
        scratch_types=[
            pltpu.SMEM((CHUNK,), jnp.int32),
            pltpu.VMEM((CHUNK, D), jnp.float32),
            pltpu.VMEM((CHUNK, D), jnp.float32),
            pltpu.VMEM_SHARED((16, b_per_w), jnp.int32),
            pltpu.SemaphoreType.DMA,
            pltpu.SemaphoreType.DMA,
            pltpu.SemaphoreType.DMA,
            pltpu.SemaphoreType.DMA,
        ],
    )
    def gather2x(
        table_hbm, idx_hbm, out_hbm,
        idx_s, rows0, rows1, idx_sh, gsem0, gsem1, wsem0, wsem1,
    ):
        wid = lax.axis_index("s") * NC + lax.axis_index("c")
        sid = lax.axis_index("s")
        base = wid * b_per_w
        pltpu.sync_copy(idx_hbm.at[pl.ds(base, b_per_w)], idx_sh.at[sid])

        def stage(j):
            pltpu.sync_copy(idx_sh.at[sid, pl.ds(j * CHUNK, CHUNK)], idx_s)

        def fire(rows_v, gsem):
            def body(r, c2):
                row = idx_s[r]
                pltpu.async_copy(
                    table_hbm.at[row >> 6, pl.ds(row & 63, 1)],
                    rows_v.at[pl.ds(r, 1)],
                    gsem,
                )
                return c2

            lax.fori_loop(0, CHUNK, body, 0, unroll=16)

        def retire(j, rows_v, gsem, wsem):
            def dbody(r, c2):
                pltpu.make_async_copy(
                    table_hbm.at[0, pl.ds(0, 1)], rows_v.at[pl.ds(r, 1)], gsem
                ).wait()
                return c2

            lax.fori_loop(0, CHUNK, dbody, 0, unroll=16)

            def mbody(r, c2):
                for c in range(D // L):
                    sl = pl.ds(c * L, L)
                    rows_v[r, sl] = rows_v[r, sl] + rows_v[r, sl]
                return c2

            lax.fori_loop(0, CHUNK, mbody, 0, unroll=4)
            pltpu.async_copy(
                rows_v, out_hbm.at[pl.ds(base + j * CHUNK, CHUNK)], wsem
            )

        def wait_writeout(rows_v, wsem):
            pltpu.make_async_copy(
                rows_v, out_hbm.at[pl.ds(base, CHUNK)], wsem
            ).wait()

        stage(0)
        fire(rows0, gsem0)

        def pair_body(k, carry):
            a = 2 * k
            b = a + 1
            stage(b)
            fire(rows1, gsem1)
            retire(a, rows0, gsem0, wsem0)

            @pl.when(b + 1 < NCHUNK)
            def _():
                stage(b + 1)
                wait_writeout(rows0, wsem0)
                fire(rows0, gsem0)

            retire(b, rows1, gsem1, wsem1)
            wait_writeout(rows1, wsem1)
            return carry

        lax.fori_loop(0, NCHUNK // 2, pair_body, 0)
        retire(NCHUNK - 1, rows0, gsem0, wsem0)
        wait_writeout(rows0, wsem0)

    return gather2x


def kernel(inputs, token_table, position_table):
    del position_table
    Bx, S = inputs.shape
    V, D = token_table.shape
    idx = inputs.reshape(-1).astype(jnp.int32)
    out = _build_gather(Bx * S, D)(token_table.reshape(V // 64, 64, D), idx)
    return out.reshape(Bx, S, D)


# speedup vs baseline: 1.0019x; 1.0019x over previous
import functools

import jax
import jax.numpy as jnp
from jax import lax
from jax.experimental import pallas as pl
from jax.experimental.pallas import tpu as pltpu
from jax.experimental.pallas import tpu_sc as plsc


def _build_gather(B: int, D: int):
    info = plsc.get_sparse_core_info()
    NC, NS, L = info.num_cores, info.num_subcores, info.num_lanes
    NW = NC * NS
    assert B % (8 * NW) == 0 and D % L == 0
    b_per_w = B // NW
    CHUNK = 256
    assert b_per_w % CHUNK == 0
    NCHUNK = b_per_w // CHUNK
    assert NCHUNK % 2 == 1

    mesh = plsc.VectorSubcoreMesh(core_axis_name="c", subcore_axis_name="s")

    @functools.partial(
        pl.kernel,
        mesh=mesh,
        compiler_params=pltpu.CompilerParams(
            use_tc_tiling_on_sc=True, needs_layout_passes=False
        ),
        out_type=jax.ShapeDtypeStruct((B, D), jnp.float32),
        scratch_types=[
            pltpu.SMEM((CHUNK,), jnp.int32),
            pltpu.VMEM((CHUNK, D), jnp.float32),
            pltpu.VMEM((CHUNK, D), jnp.float32),
            pltpu.VMEM_SHARED((16, b_per_w), jnp.int32),
            pltpu.SemaphoreType.DMA,
            pltpu.SemaphoreType.DMA,
            pltpu.SemaphoreType.DMA,
            pltpu.SemaphoreType.DMA,
        ],
    )
    def gather2x(
        table_hbm, idx_hbm, out_hbm,
        idx_s, rows0, rows1, idx_sh, gsem0, gsem1, wsem0, wsem1,
    ):
        wid = lax.axis_index("s") * NC + lax.axis_index("c")
        sid = lax.axis_index("s")
        base = wid * b_per_w
        pltpu.sync_copy(idx_hbm.at[pl.ds(base, b_per_w)], idx_sh.at[sid])

        def stage(j):
            pltpu.sync_copy(idx_sh.at[sid, pl.ds(j * CHUNK, CHUNK)], idx_s)

        def fire(rows_v, gsem):
            def body(r, c2):
                row = idx_s[r]
                pltpu.async_copy(
                    table_hbm.at[row >> 6, pl.ds(row & 63, 1)],
                    rows_v.at[pl.ds(r, 1)],
                    gsem,
                )
                return c2

            lax.fori_loop(0, CHUNK, body, 0, unroll=8)

        def retire(j, rows_v, gsem, wsem):
            def dbody(r, c2):
                pltpu.make_async_copy(
                    table_hbm.at[0, pl.ds(0, 1)], rows_v.at[pl.ds(r, 1)], gsem
                ).wait()
                return c2

            lax.fori_loop(0, CHUNK, dbody, 0, unroll=8)

            def mbody(r, c2):
                for c in range(D // L):
                    sl = pl.ds(c * L, L)
                    rows_v[r, sl] = rows_v[r, sl] + rows_v[r, sl]
                return c2

            lax.fori_loop(0, CHUNK, mbody, 0, unroll=4)
            pltpu.async_copy(
                rows_v, out_hbm.at[pl.ds(base + j * CHUNK, CHUNK)], wsem
            )

        def wait_writeout(rows_v, wsem):
            pltpu.make_async_copy(
                rows_v, out_hbm.at[pl.ds(base, CHUNK)], wsem
            ).wait()

        stage(0)
        fire(rows0, gsem0)

        def pair_body(k, carry):
            a = 2 * k
            b = a + 1
            stage(b)
            fire(rows1, gsem1)
            retire(a, rows0, gsem0, wsem0)

            @pl.when(b + 1 < NCHUNK)
            def _():
                stage(b + 1)
                wait_writeout(rows0, wsem0)
                fire(rows0, gsem0)

            retire(b, rows1, gsem1, wsem1)
            wait_writeout(rows1, wsem1)
            return carry

        lax.fori_loop(0, NCHUNK // 2, pair_body, 0)
        retire(NCHUNK - 1, rows0, gsem0, wsem0)
        wait_writeout(rows0, wsem0)

    return gather2x


def kernel(inputs, token_table, position_table):
    del position_table
    Bx, S = inputs.shape
    V, D = token_table.shape
    idx = inputs.reshape(-1).astype(jnp.int32)
    out = _build_gather(Bx * S, D)(token_table.reshape(V // 64, 64, D), idx)
    return out.reshape(Bx, S, D)


# speedup vs baseline: 1.0499x; 1.0480x over previous
import functools

import jax
import jax.numpy as jnp
from jax import lax
from jax.experimental import pallas as pl
from jax.experimental.pallas import tpu as pltpu
from jax.experimental.pallas import tpu_sc as plsc


def _build_gather(B: int, D: int):
    info = plsc.get_sparse_core_info()
    NC, NS, L = info.num_cores, info.num_subcores, info.num_lanes
    NW = NC * NS
    assert B % (8 * NW) == 0 and D % L == 0
    b_per_w = B // NW
    CHUNK = 256
    assert b_per_w % CHUNK == 0
    NCHUNK = b_per_w // CHUNK
    assert NCHUNK % 2 == 1

    mesh = plsc.VectorSubcoreMesh(core_axis_name="c", subcore_axis_name="s")

    @functools.partial(
        pl.kernel,
        mesh=mesh,
        compiler_params=pltpu.CompilerParams(
            use_tc_tiling_on_sc=True, needs_layout_passes=False
        ),
        out_type=jax.ShapeDtypeStruct((B, D), jnp.float32),
        scratch_types=[
            pltpu.SMEM((CHUNK,), jnp.int32),
            pltpu.VMEM((CHUNK, D), jnp.float32),
            pltpu.VMEM((CHUNK, D), jnp.float32),
            pltpu.VMEM_SHARED((16, b_per_w), jnp.int32),
            pltpu.SemaphoreType.DMA,
            pltpu.SemaphoreType.DMA,
            pltpu.SemaphoreType.DMA,
            pltpu.SemaphoreType.DMA,
        ],
    )
    def gather2x(
        table_hbm, idx_hbm, out_hbm,
        idx_s, rows0, rows1, idx_sh, gsem0, gsem1, wsem0, wsem1,
    ):
        wid = lax.axis_index("s") * NC + lax.axis_index("c")
        sid = lax.axis_index("s")
        base = wid * b_per_w
        pltpu.sync_copy(idx_hbm.at[pl.ds(base, b_per_w)], idx_sh.at[sid])

        def stage(j):
            pltpu.sync_copy(idx_sh.at[sid, pl.ds(j * CHUNK, CHUNK)], idx_s)

        def fire(rows_v, gsem):
            def body(r, c2):
                row = idx_s[r]
                pltpu.async_copy(
                    table_hbm.at[row >> 6, pl.ds(row & 63, 1)],
                    rows_v.at[pl.ds(r, 1)],
                    gsem,
                )
                return c2

            lax.fori_loop(0, CHUNK, body, 0, unroll=8)

        def retire(j, rows_v, gsem, wsem):
            pltpu.make_async_copy(
                out_hbm.at[pl.ds(base, CHUNK)], rows_v, gsem
            ).wait()

            def mbody(r, c2):
                for c in range(D // L):
                    sl = pl.ds(c * L, L)
                    rows_v[r, sl] = rows_v[r, sl] + rows_v[r, sl]
                return c2

            lax.fori_loop(0, CHUNK, mbody, 0, unroll=4)
            pltpu.async_copy(
                rows_v, out_hbm.at[pl.ds(base + j * CHUNK, CHUNK)], wsem
            )

        def wait_writeout(rows_v, wsem):
            pltpu.make_async_copy(
                rows_v, out_hbm.at[pl.ds(base, CHUNK)], wsem
            ).wait()

        stage(0)
        fire(rows0, gsem0)

        def pair_body(k, carry):
            a = 2 * k
            b = a + 1
            stage(b)
            fire(rows1, gsem1)
            retire(a, rows0, gsem0, wsem0)

            @pl.when(b + 1 < NCHUNK)
            def _():
                stage(b + 1)
                wait_writeout(rows0, wsem0)
                fire(rows0, gsem0)

            retire(b, rows1, gsem1, wsem1)
            wait_writeout(rows1, wsem1)
            return carry

        lax.fori_loop(0, NCHUNK // 2, pair_body, 0)
        retire(NCHUNK - 1, rows0, gsem0, wsem0)
        wait_writeout(rows0, wsem0)

    return gather2x


def kernel(inputs, token_table, position_table):
    del position_table
    Bx, S = inputs.shape
    V, D = token_table.shape
    idx = inputs.reshape(-1).astype(jnp.int32)
    out = _build_gather(Bx * S, D)(token_table.reshape(V // 64, 64, D), idx)
    return out.reshape(Bx, S, D)
